# baseline (device time: 36963 ns/iter reference)
import jax
import jax.numpy as jnp
from jax import lax
from jax.experimental import pallas as pl
from jax.experimental.pallas import tpu as pltpu

N_DEV = 4
N_LAYERS = 3
N_COL = 2

def _slot(stage, half, col):
    return (stage * 2 + half) * N_COL + col

N_SLOTS = 3 * 2 * N_COL


def kernel(x, Win0, Wout0, Win1, Wout1, Win2, Wout2):
    b, _ = x.shape
    h_dim = Win0.shape[1]
    half = b // 2
    sub = b // 4
    cw = h_dim // N_COL

    def body(x_ref, win0_ref, wout0_ref, win1_ref, wout1_ref, win2_ref,
             wout2_ref, out_ref, acc_ref, h_ref, xbuf_ref, hbf_ref,
             tx1_buf, tx2_buf, s1_buf, s2_buf, send_sems, recv_sems):
        my = lax.axis_index("i")
        pa = my ^ 1
        pb = 3 - my
        g = (my ^ (my >> 1)) & 1
        rb = my >> 1

        kA = g * sub
        sA = (1 - g) * sub
        kB = half + rb * sub
        sB = half + (1 - rb) * sub
        keep = (kA, kB)
        sent = (sA, sB)
        p1 = (pa, pb)
        p2 = (pb, pa)

        barrier_sem = pltpu.get_barrier_semaphore()
        for nbr in (pa, pb):
            pl.semaphore_signal(
                barrier_sem, inc=1,
                device_id=(nbr,), device_id_type=pl.DeviceIdType.MESH,
            )
        pl.semaphore_wait(barrier_sem, 2)

        def exch(stage, hf, col, src_ref, dst_ref, peer):
            sl = _slot(stage, hf, col)
            rdma = pltpu.make_async_remote_copy(
                src_ref=src_ref,
                dst_ref=dst_ref,
                send_sem=send_sems.at[sl],
                recv_sem=recv_sems.at[sl],
                device_id=(peer,),
                device_id_type=pl.DeviceIdType.MESH,
            )
            rdma.start()
            return rdma

        win_refs = [win0_ref, win1_ref, win2_ref]
        wout_refs = [wout0_ref, wout1_ref, wout2_ref]

        for layer in range(N_LAYERS):
            xsrc = x_ref if layer == 0 else xbuf_ref
            xdst = out_ref if layer == N_LAYERS - 1 else xbuf_ref
            win = win_refs[layer]
            wout = wout_refs[layer]

            r1 = {}
            for col in range(N_COL):
                cols = pl.ds(col * cw, cw)
                for hf in range(2):
                    tx1_buf[hf, :, cols] = jnp.dot(
                        xsrc[pl.ds(sent[hf], sub), :], win[:, cols],
                        preferred_element_type=jnp.float32,
                    ).astype(jnp.bfloat16)
                    r1[hf, col] = exch(
                        0, hf, col,
                        tx1_buf.at[hf, :, cols], s1_buf.at[hf, :, cols],
                        p1[hf],
                    )
            for col in range(N_COL):
                cols = pl.ds(col * cw, cw)
                for hf in range(2):
                    acc_ref[pl.ds(keep[hf], sub), cols] = jnp.dot(
                        xsrc[pl.ds(keep[hf], sub), :], win[:, cols],
                        preferred_element_type=jnp.float32,
                    )

            r2 = {}
            for col in range(N_COL):
                cols = pl.ds(col * cw, cw)
                for hf in range(2):
                    r1[hf, col].wait_recv()
                    rows = pl.ds(keep[hf], sub)
                    summed = acc_ref[rows, cols] + s1_buf[hf, :, cols].astype(
                        jnp.float32
                    )
                    acc_ref[rows, cols] = summed
                    tx2_buf[hf, :, cols] = summed.astype(jnp.bfloat16)
                    r2[hf, col] = exch(
                        1, hf, col,
                        tx2_buf.at[hf, :, cols], s2_buf.at[hf, :, cols],
                        p2[hf],
                    )
            r3 = {}
            for col in range(N_COL):
                cols = pl.ds(col * cw, cw)
                for hf in range(2):
                    r2[hf, col].wait_recv()
                    rows = pl.ds(keep[hf], sub)
                    hred = jnp.maximum(
                        acc_ref[rows, cols] + s2_buf[hf, :, cols].astype(
                            jnp.float32
                        ),
                        0.0,
                    )
                    h_ref[rows, cols] = hred
                    hbf_ref[rows, cols] = hred.astype(jnp.bfloat16)
                    r3[hf, col] = exch(
                        2, hf, col,
                        hbf_ref.at[rows, cols], hbf_ref.at[rows, cols],
                        p1[hf],
                    )

            for hf in range(2):
                xdst[pl.ds(keep[hf], sub), :] = jnp.dot(
                    h_ref[pl.ds(keep[hf], sub), :], wout[...],
                    preferred_element_type=jnp.float32,
                )
            for col in range(N_COL):
                cols = pl.ds(col * cw, cw)
                for hf in range(2):
                    r3[hf, col].wait_recv()
                    rows = pl.ds(sent[hf], sub)
                    contrib = jnp.dot(
                        hbf_ref[rows, cols].astype(jnp.float32),
                        wout[cols, :],
                        preferred_element_type=jnp.float32,
                    )
                    if col == 0:
                        xdst[rows, :] = contrib
                    else:
                        xdst[rows, :] = xdst[rows, :] + contrib
            for r in list(r1.values()) + list(r2.values()) + list(r3.values()):
                r.wait_send()

    return pl.pallas_call(
        body,
        out_shape=jax.ShapeDtypeStruct(x.shape, jnp.float32),
        in_specs=[pl.BlockSpec(memory_space=pltpu.VMEM)] * 7,
        out_specs=pl.BlockSpec(memory_space=pltpu.VMEM),
        scratch_shapes=[
            pltpu.VMEM((b, h_dim), jnp.float32),
            pltpu.VMEM((b, h_dim), jnp.float32),
            pltpu.VMEM(x.shape, jnp.float32),
            pltpu.VMEM((b, h_dim), jnp.bfloat16),
            pltpu.VMEM((2, sub, h_dim), jnp.bfloat16),
            pltpu.VMEM((2, sub, h_dim), jnp.bfloat16),
            pltpu.VMEM((2, sub, h_dim), jnp.bfloat16),
            pltpu.VMEM((2, sub, h_dim), jnp.bfloat16),
            pltpu.SemaphoreType.DMA((N_SLOTS,)),
            pltpu.SemaphoreType.DMA((N_SLOTS,)),
        ],
        compiler_params=pltpu.CompilerParams(collective_id=0),
    )(x, Win0, Wout0, Win1, Wout1, Win2, Wout2)


# device time: 35107 ns/iter; 1.0529x vs baseline; 1.0529x over previous
import jax
import jax.numpy as jnp
from jax import lax
from jax.experimental import pallas as pl
from jax.experimental.pallas import tpu as pltpu

N_DEV = 4
N_LAYERS = 3
N_COL = 4

def _slot(stage, half, col):
    return (stage * 2 + half) * N_COL + col

N_SLOTS = 3 * 2 * N_COL


def kernel(x, Win0, Wout0, Win1, Wout1, Win2, Wout2):
    b, _ = x.shape
    h_dim = Win0.shape[1]
    half = b // 2
    sub = b // 4
    cw = h_dim // N_COL

    def body(x_ref, win0_ref, wout0_ref, win1_ref, wout1_ref, win2_ref,
             wout2_ref, out_ref, acc_ref, h_ref, xbuf_ref, hbf_ref,
             tx1_buf, tx2_buf, s1_buf, s2_buf, send_sems, recv_sems):
        my = lax.axis_index("i")
        pa = my ^ 1
        pb = 3 - my
        g = (my ^ (my >> 1)) & 1
        rb = my >> 1

        kA = g * sub
        sA = (1 - g) * sub
        kB = half + rb * sub
        sB = half + (1 - rb) * sub
        keep = (kA, kB)
        sent = (sA, sB)
        p1 = (pa, pb)
        p2 = (pb, pa)

        barrier_sem = pltpu.get_barrier_semaphore()
        for nbr in (pa, pb):
            pl.semaphore_signal(
                barrier_sem, inc=1,
                device_id=(nbr,), device_id_type=pl.DeviceIdType.MESH,
            )
        pl.semaphore_wait(barrier_sem, 2)

        def exch(stage, hf, col, src_ref, dst_ref, peer):
            sl = _slot(stage, hf, col)
            rdma = pltpu.make_async_remote_copy(
                src_ref=src_ref,
                dst_ref=dst_ref,
                send_sem=send_sems.at[sl],
                recv_sem=recv_sems.at[sl],
                device_id=(peer,),
                device_id_type=pl.DeviceIdType.MESH,
            )
            rdma.start()
            return rdma

        win_refs = [win0_ref, win1_ref, win2_ref]
        wout_refs = [wout0_ref, wout1_ref, wout2_ref]

        for layer in range(N_LAYERS):
            xsrc = x_ref if layer == 0 else xbuf_ref
            xdst = out_ref if layer == N_LAYERS - 1 else xbuf_ref
            win = win_refs[layer]
            wout = wout_refs[layer]

            r1 = {}
            for col in range(N_COL):
                cols = pl.ds(col * cw, cw)
                for hf in range(2):
                    tx1_buf[hf, :, cols] = jnp.dot(
                        xsrc[pl.ds(sent[hf], sub), :], win[:, cols],
                        preferred_element_type=jnp.float32,
                    ).astype(jnp.bfloat16)
                    r1[hf, col] = exch(
                        0, hf, col,
                        tx1_buf.at[hf, :, cols], s1_buf.at[hf, :, cols],
                        p1[hf],
                    )
            for hf in range(2):
                acc_ref[pl.ds(keep[hf], sub), :] = jnp.dot(
                    xsrc[pl.ds(keep[hf], sub), :], win[...],
                    preferred_element_type=jnp.float32,
                )

            r2 = {}
            for col in range(N_COL):
                cols = pl.ds(col * cw, cw)
                for hf in range(2):
                    r1[hf, col].wait_recv()
                    rows = pl.ds(keep[hf], sub)
                    summed = acc_ref[rows, cols] + s1_buf[hf, :, cols].astype(
                        jnp.float32
                    )
                    acc_ref[rows, cols] = summed
                    tx2_buf[hf, :, cols] = summed.astype(jnp.bfloat16)
                    r2[hf, col] = exch(
                        1, hf, col,
                        tx2_buf.at[hf, :, cols], s2_buf.at[hf, :, cols],
                        p2[hf],
                    )
            r3 = {}
            for col in range(N_COL):
                cols = pl.ds(col * cw, cw)
                for hf in range(2):
                    r2[hf, col].wait_recv()
                    rows = pl.ds(keep[hf], sub)
                    hred = jnp.maximum(
                        acc_ref[rows, cols] + s2_buf[hf, :, cols].astype(
                            jnp.float32
                        ),
                        0.0,
                    )
                    h_ref[rows, cols] = hred
                    hbf_ref[rows, cols] = hred.astype(jnp.bfloat16)
                    r3[hf, col] = exch(
                        2, hf, col,
                        hbf_ref.at[rows, cols], hbf_ref.at[rows, cols],
                        p1[hf],
                    )

            for hf in range(2):
                xdst[pl.ds(keep[hf], sub), :] = jnp.dot(
                    h_ref[pl.ds(keep[hf], sub), :], wout[...],
                    preferred_element_type=jnp.float32,
                )
            for col in range(N_COL):
                cols = pl.ds(col * cw, cw)
                for hf in range(2):
                    r3[hf, col].wait_recv()
                    rows = pl.ds(sent[hf], sub)
                    contrib = jnp.dot(
                        hbf_ref[rows, cols].astype(jnp.float32),
                        wout[cols, :],
                        preferred_element_type=jnp.float32,
                    )
                    if col == 0:
                        xdst[rows, :] = contrib
                    else:
                        xdst[rows, :] = xdst[rows, :] + contrib
            for r in list(r1.values()) + list(r2.values()) + list(r3.values()):
                r.wait_send()

    return pl.pallas_call(
        body,
        out_shape=jax.ShapeDtypeStruct(x.shape, jnp.float32),
        in_specs=[pl.BlockSpec(memory_space=pltpu.VMEM)] * 7,
        out_specs=pl.BlockSpec(memory_space=pltpu.VMEM),
        scratch_shapes=[
            pltpu.VMEM((b, h_dim), jnp.float32),
            pltpu.VMEM((b, h_dim), jnp.float32),
            pltpu.VMEM(x.shape, jnp.float32),
            pltpu.VMEM((b, h_dim), jnp.bfloat16),
            pltpu.VMEM((2, sub, h_dim), jnp.bfloat16),
            pltpu.VMEM((2, sub, h_dim), jnp.bfloat16),
            pltpu.VMEM((2, sub, h_dim), jnp.bfloat16),
            pltpu.VMEM((2, sub, h_dim), jnp.bfloat16),
            pltpu.SemaphoreType.DMA((N_SLOTS,)),
            pltpu.SemaphoreType.DMA((N_SLOTS,)),
        ],
        compiler_params=pltpu.CompilerParams(collective_id=0),
    )(x, Win0, Wout0, Win1, Wout1, Win2, Wout2)
